# baseline (speedup 1.0000x reference)
import functools

import jax
import jax.numpy as jnp
from jax import lax
from jax.experimental import pallas as pl
from jax.experimental.pallas import tpu as pltpu
from jax.experimental.pallas import tpu_sc as plsc

N = 4096
D = 16
KNN = 8
BLK = 256
NBLK = N // BLK

NC = 2
NS = 16
NW = NC * NS
LANES = 16
EDGES = N * KNN
E_PER_W = EDGES // NW
GROUPS = E_PER_W // LANES
ROWS_PER_W = N // NW


def _knn_kernel(xb_ref, xa_ref, sqb_ref, sqt_ref, d_ref, i_ref, scratch):
    r0 = pl.program_id(0) * BLK
    dot = jax.lax.dot_general(
        xb_ref[...], xa_ref[...], (((1,), (1,)), ((), ())),
        preferred_element_type=jnp.float32,
    )
    d2 = (sqb_ref[...] + sqt_ref[...]) - 2.0 * dot
    d2 = jnp.maximum(d2, 0.0)
    col = jax.lax.broadcasted_iota(jnp.int32, (BLK, N), 1)
    row = jax.lax.broadcasted_iota(jnp.int32, (BLK, N), 0) + r0
    colf = col.astype(jnp.float32)
    inf = jnp.float32(jnp.inf)
    scratch[...] = jnp.where(col == row, inf, d2)
    for k in range(KNN):
        cur = scratch[...]
        mv = jnp.min(cur, axis=1, keepdims=True)
        aminf = jnp.min(jnp.where(cur == mv, colf, jnp.float32(N)),
                        axis=1, keepdims=True)
        d_ref[:, k : k + 1] = mv
        i_ref[:, k : k + 1] = aminf.astype(jnp.int32)
        if k + 1 < KNN:
            scratch[...] = jnp.where(colf == aminf, inf, cur)


def _weights_kernel(knn_ref, w_ref):
    knn = knn_ref[...]
    sigma = jnp.sum(knn) / jnp.float32(N * KNN) + jnp.float32(1e-8)
    w_ref[...] = jnp.exp(-knn / sigma)


def _sc_edges_kernel(idx_hbm, w_hbm, pdeg_hbm, offs_hbm, vals_hbm,
                     idx_v, w_v, pdeg_v, offs_v, vals_v):
    wid = lax.axis_index("s") * NC + lax.axis_index("c")
    pltpu.sync_copy(idx_hbm, idx_v)
    pltpu.sync_copy(w_hbm, w_v)
    zero16 = jnp.zeros((LANES,), jnp.float32)

    @pl.loop(0, N // LANES)
    def _(i):
        pdeg_v[pl.ds(i * LANES, LANES)] = zero16

    base_e = wid * E_PER_W
    lane = lax.iota(jnp.int32, LANES)

    @pl.loop(0, GROUPS)
    def _(g):
        e0 = base_e + g * LANES
        j = (e0 + lane) >> 3
        t = idx_v[pl.ds(e0, LANES)]
        wv = w_v[pl.ds(e0, LANES)]
        m = plsc.load_gather(idx_v, [t * KNN]) == j
        for kk in range(1, KNN):
            m = jnp.logical_or(m, plsc.load_gather(idx_v, [t * KNN + kk]) == j)
        plsc.addupdate_scatter(pdeg_v, [j], wv)
        plsc.addupdate_scatter(pdeg_v, [t], jnp.where(m, 0.0, wv))
        r = g >> 2
        c = (g & 3) * (2 * LANES)
        offs_v[r, pl.ds(c, LANES)] = j * N + t
        offs_v[r, pl.ds(c + LANES, LANES)] = t * N + j
        vals_v[r, pl.ds(c, LANES)] = -wv
        vals_v[r, pl.ds(c + LANES, LANES)] = -wv

    pltpu.sync_copy(pdeg_v, pdeg_hbm.at[wid])
    pltpu.sync_copy(offs_v, offs_hbm.at[wid])
    pltpu.sync_copy(vals_v, vals_hbm.at[wid])


def _sc_scatter_kernel(l_ref, pdeg_hbm, offs_hbm, vals_hbm,
                       offs_v, vals_v, acc_v, tmp_v, sem):
    wid = lax.axis_index("s") * NC + lax.axis_index("c")
    pltpu.sync_copy(offs_hbm.at[wid], offs_v.at[pl.ds(0, 16)])
    pltpu.sync_copy(vals_hbm.at[wid], vals_v.at[pl.ds(0, 16)])
    row0 = wid * ROWS_PER_W
    zero16 = jnp.zeros((LANES,), jnp.float32)
    for c in range(ROWS_PER_W // LANES):
        acc_v[pl.ds(c * LANES, LANES)] = zero16

    @pl.loop(0, NW)
    def _(s):
        pltpu.sync_copy(pdeg_hbm.at[s, pl.ds(row0, ROWS_PER_W)], tmp_v)
        for c in range(ROWS_PER_W // LANES):
            sl = pl.ds(c * LANES, LANES)
            acc_v[sl] = acc_v[sl] + tmp_v[sl]

    lane = lax.iota(jnp.int32, LANES)
    for c in range(ROWS_PER_W // LANES):
        rr = row0 + c * LANES + lane
        offs_v[16, pl.ds(c * LANES, LANES)] = rr * (N + 1)
        vals_v[16, pl.ds(c * LANES, LANES)] = acc_v[pl.ds(c * LANES, LANES)]

    copies = [
        pltpu.async_copy(vals_v.at[r], l_ref.at[offs_v.at[r]], sem)
        for r in range(17)
    ]
    for cp in copies:
        cp.wait()


_sc_edges = functools.partial(
    pl.kernel,
    out_type=[
        jax.ShapeDtypeStruct((NW, N), jnp.float32),
        jax.ShapeDtypeStruct((NW, 16, 128), jnp.int32),
        jax.ShapeDtypeStruct((NW, 16, 128), jnp.float32),
    ],
    mesh=plsc.VectorSubcoreMesh(core_axis_name="c", subcore_axis_name="s"),
    compiler_params=pltpu.CompilerParams(needs_layout_passes=False),
    scratch_types=[
        pltpu.VMEM((EDGES,), jnp.int32),
        pltpu.VMEM((EDGES,), jnp.float32),
        pltpu.VMEM((N,), jnp.float32),
        pltpu.VMEM((16, 128), jnp.int32),
        pltpu.VMEM((16, 128), jnp.float32),
    ],
)(_sc_edges_kernel)

_sc_scatter = functools.partial(
    pl.kernel,
    out_type=(),
    mesh=plsc.VectorSubcoreMesh(core_axis_name="c", subcore_axis_name="s"),
    compiler_params=pltpu.CompilerParams(needs_layout_passes=False),
    scratch_types=[
        pltpu.VMEM((17, 128), jnp.int32),
        pltpu.VMEM((17, 128), jnp.float32),
        pltpu.VMEM((ROWS_PER_W,), jnp.float32),
        pltpu.VMEM((ROWS_PER_W,), jnp.float32),
        pltpu.SemaphoreType.DMA,
    ],
)(_sc_scatter_kernel)


def _assemble_kernel(knn_ref, idx_ref, knn_t_ref, idx_t_ref, out_ref):
    r0 = pl.program_id(0) * BLK
    knn_t = knn_t_ref[...]
    sigma = jnp.sum(knn_t) / jnp.float32(N * KNN) + jnp.float32(1e-8)
    w_t = jnp.exp(-knn_t / sigma)
    w_b = jnp.exp(-knn_ref[...] / sigma)
    idx_b = idx_ref[...]

    col = jax.lax.broadcasted_iota(jnp.int32, (BLK, N), 1)
    row = jax.lax.broadcasted_iota(jnp.int32, (BLK, N), 0) + r0
    acc = jnp.zeros((BLK, N), jnp.float32)
    for k in range(KNN):
        own = col == idx_b[:, k : k + 1]
        acc = jnp.where(own, w_b[:, k : k + 1], acc)
        trn = idx_t_ref[k : k + 1, :] == row
        acc = jnp.where(trn, w_t[k : k + 1, :], acc)
    deg = jnp.sum(acc, axis=1, keepdims=True)
    out_ref[...] = jnp.where(col == row, deg, -acc)


@jax.jit
def kernel(x):
    sq = jnp.sum(x * x, axis=1)
    sq_col = sq[:, None]
    sq_row = sq[None, :]

    knn_d2, idx = pl.pallas_call(
        _knn_kernel,
        grid=(NBLK,),
        in_specs=[
            pl.BlockSpec((BLK, D), lambda i: (i, 0)),
            pl.BlockSpec((N, D), lambda i: (0, 0)),
            pl.BlockSpec((BLK, 1), lambda i: (i, 0)),
            pl.BlockSpec((1, N), lambda i: (0, 0)),
        ],
        out_specs=[
            pl.BlockSpec((BLK, KNN), lambda i: (i, 0)),
            pl.BlockSpec((BLK, KNN), lambda i: (i, 0)),
        ],
        out_shape=[
            jax.ShapeDtypeStruct((N, KNN), jnp.float32),
            jax.ShapeDtypeStruct((N, KNN), jnp.int32),
        ],
        scratch_shapes=[pltpu.VMEM((BLK, N), jnp.float32)],
    )(x, x, sq_col, sq_row)

    w = pl.pallas_call(
        _weights_kernel,
        out_shape=jax.ShapeDtypeStruct((N, KNN), jnp.float32),
    )(knn_d2)

    idx_flat = idx.reshape(EDGES)
    w_flat = w.reshape(EDGES)
    pdeg, offs, vals = _sc_edges(idx_flat, w_flat)

    l_ref = jax.new_ref(jnp.zeros((N * N,), jnp.float32))
    _sc_scatter(l_ref, pdeg, offs, vals)
    return l_ref[...].reshape(N, N)


# speedup vs baseline: 1.0366x; 1.0366x over previous
import functools

import jax
import jax.numpy as jnp
from jax import lax
from jax.experimental import pallas as pl
from jax.experimental.pallas import tpu as pltpu
from jax.experimental.pallas import tpu_sc as plsc

N = 4096
D = 16
KNN = 8
BLK = 256
NBLK = N // BLK

NC = 2
NS = 16
NW = NC * NS
LANES = 16
EDGES = N * KNN
E_PER_W = EDGES // NW
GROUPS = E_PER_W // LANES
ROWS_PER_W = N // NW


def _knn_kernel(xb_ref, xa_ref, sqb_ref, sqt_ref, d_ref, i_ref, scratch):
    r0 = pl.program_id(0) * BLK
    dot = jax.lax.dot_general(
        xb_ref[...], xa_ref[...], (((1,), (1,)), ((), ())),
        preferred_element_type=jnp.float32,
    )
    d2 = (sqb_ref[...] + sqt_ref[...]) - 2.0 * dot
    d2 = jnp.maximum(d2, 0.0)
    col = jax.lax.broadcasted_iota(jnp.int32, (BLK, N), 1)
    row = jax.lax.broadcasted_iota(jnp.int32, (BLK, N), 0) + r0
    colf = col.astype(jnp.float32)
    inf = jnp.float32(jnp.inf)
    scratch[...] = jnp.where(col == row, inf, d2)
    for k in range(KNN):
        cur = scratch[...]
        mv = jnp.min(cur, axis=1, keepdims=True)
        aminf = jnp.min(jnp.where(cur == mv, colf, jnp.float32(N)),
                        axis=1, keepdims=True)
        d_ref[:, k : k + 1] = mv
        i_ref[:, k : k + 1] = aminf.astype(jnp.int32)
        if k + 1 < KNN:
            scratch[...] = jnp.where(colf == aminf, inf, cur)


def _weights_kernel(knn_ref, w_ref):
    knn = knn_ref[...]
    sigma = jnp.sum(knn) / jnp.float32(N * KNN) + jnp.float32(1e-8)
    w_ref[...] = jnp.exp(-knn / sigma)


def _sc_edges_kernel(idx_hbm, w_hbm, pdeg_hbm, offs_hbm, vals_hbm,
                     idx_v, w_v, pdeg_v, offs_v, vals_v):
    wid = lax.axis_index("s") * NC + lax.axis_index("c")
    pltpu.sync_copy(idx_hbm, idx_v)
    pltpu.sync_copy(w_hbm, w_v)
    zero16 = jnp.zeros((LANES,), jnp.float32)

    @pl.loop(0, N // LANES)
    def _(i):
        pdeg_v[pl.ds(i * LANES, LANES)] = zero16

    base_e = wid * E_PER_W
    lane = lax.iota(jnp.int32, LANES)

    @pl.loop(0, GROUPS)
    def _(g):
        e0 = base_e + g * LANES
        j = (e0 + lane) >> 3
        t = idx_v[pl.ds(e0, LANES)]
        wv = w_v[pl.ds(e0, LANES)]
        m = plsc.load_gather(idx_v, [t * KNN]) == j
        for kk in range(1, KNN):
            m = jnp.logical_or(m, plsc.load_gather(idx_v, [t * KNN + kk]) == j)
        plsc.addupdate_scatter(pdeg_v, [j], wv)
        plsc.addupdate_scatter(pdeg_v, [t], jnp.where(m, 0.0, wv))
        r = g >> 2
        c = (g & 3) * (2 * LANES)
        offs_v[r, pl.ds(c, LANES)] = j * N + t
        offs_v[r, pl.ds(c + LANES, LANES)] = t * N + j
        vals_v[r, pl.ds(c, LANES)] = -wv
        vals_v[r, pl.ds(c + LANES, LANES)] = -wv

    pltpu.sync_copy(pdeg_v, pdeg_hbm.at[wid])
    pltpu.sync_copy(offs_v, offs_hbm.at[wid])
    pltpu.sync_copy(vals_v, vals_hbm.at[wid])


def _sc_scatter_kernel(l_ref, pdeg_hbm, offs_hbm, vals_hbm,
                       offs_v, vals_v, tmp_v, sem):
    wid = lax.axis_index("s") * NC + lax.axis_index("c")
    pltpu.sync_copy(offs_hbm.at[wid], offs_v.at[pl.ds(0, 16)])
    pltpu.sync_copy(vals_hbm.at[wid], vals_v.at[pl.ds(0, 16)])
    row0 = wid * ROWS_PER_W
    pltpu.sync_copy(pdeg_hbm.at[:, pl.ds(row0, ROWS_PER_W)], tmp_v)

    lane = lax.iota(jnp.int32, LANES)
    for c in range(ROWS_PER_W // LANES):
        sl = pl.ds(c * LANES, LANES)
        acc = tmp_v[0, sl]
        for s in range(1, NW):
            acc = acc + tmp_v[s, sl]
        rr = row0 + c * LANES + lane
        offs_v[16, sl] = rr * (N + 1)
        vals_v[16, sl] = acc

    copies = [
        pltpu.async_copy(vals_v.at[r], l_ref.at[offs_v.at[r]], sem)
        for r in range(17)
    ]
    for cp in copies:
        cp.wait()


_sc_edges = functools.partial(
    pl.kernel,
    out_type=[
        jax.ShapeDtypeStruct((NW, N), jnp.float32),
        jax.ShapeDtypeStruct((NW, 16, 128), jnp.int32),
        jax.ShapeDtypeStruct((NW, 16, 128), jnp.float32),
    ],
    mesh=plsc.VectorSubcoreMesh(core_axis_name="c", subcore_axis_name="s"),
    compiler_params=pltpu.CompilerParams(needs_layout_passes=False),
    scratch_types=[
        pltpu.VMEM((EDGES,), jnp.int32),
        pltpu.VMEM((EDGES,), jnp.float32),
        pltpu.VMEM((N,), jnp.float32),
        pltpu.VMEM((16, 128), jnp.int32),
        pltpu.VMEM((16, 128), jnp.float32),
    ],
)(_sc_edges_kernel)

_sc_scatter = functools.partial(
    pl.kernel,
    out_type=(),
    mesh=plsc.VectorSubcoreMesh(core_axis_name="c", subcore_axis_name="s"),
    compiler_params=pltpu.CompilerParams(needs_layout_passes=False),
    scratch_types=[
        pltpu.VMEM((17, 128), jnp.int32),
        pltpu.VMEM((17, 128), jnp.float32),
        pltpu.VMEM((NW, ROWS_PER_W), jnp.float32),
        pltpu.SemaphoreType.DMA,
    ],
)(_sc_scatter_kernel)


def _assemble_kernel(knn_ref, idx_ref, knn_t_ref, idx_t_ref, out_ref):
    r0 = pl.program_id(0) * BLK
    knn_t = knn_t_ref[...]
    sigma = jnp.sum(knn_t) / jnp.float32(N * KNN) + jnp.float32(1e-8)
    w_t = jnp.exp(-knn_t / sigma)
    w_b = jnp.exp(-knn_ref[...] / sigma)
    idx_b = idx_ref[...]

    col = jax.lax.broadcasted_iota(jnp.int32, (BLK, N), 1)
    row = jax.lax.broadcasted_iota(jnp.int32, (BLK, N), 0) + r0
    acc = jnp.zeros((BLK, N), jnp.float32)
    for k in range(KNN):
        own = col == idx_b[:, k : k + 1]
        acc = jnp.where(own, w_b[:, k : k + 1], acc)
        trn = idx_t_ref[k : k + 1, :] == row
        acc = jnp.where(trn, w_t[k : k + 1, :], acc)
    deg = jnp.sum(acc, axis=1, keepdims=True)
    out_ref[...] = jnp.where(col == row, deg, -acc)


@jax.jit
def kernel(x):
    sq = jnp.sum(x * x, axis=1)
    sq_col = sq[:, None]
    sq_row = sq[None, :]

    knn_d2, idx = pl.pallas_call(
        _knn_kernel,
        grid=(NBLK,),
        in_specs=[
            pl.BlockSpec((BLK, D), lambda i: (i, 0)),
            pl.BlockSpec((N, D), lambda i: (0, 0)),
            pl.BlockSpec((BLK, 1), lambda i: (i, 0)),
            pl.BlockSpec((1, N), lambda i: (0, 0)),
        ],
        out_specs=[
            pl.BlockSpec((BLK, KNN), lambda i: (i, 0)),
            pl.BlockSpec((BLK, KNN), lambda i: (i, 0)),
        ],
        out_shape=[
            jax.ShapeDtypeStruct((N, KNN), jnp.float32),
            jax.ShapeDtypeStruct((N, KNN), jnp.int32),
        ],
        scratch_shapes=[pltpu.VMEM((BLK, N), jnp.float32)],
    )(x, x, sq_col, sq_row)

    w = pl.pallas_call(
        _weights_kernel,
        out_shape=jax.ShapeDtypeStruct((N, KNN), jnp.float32),
    )(knn_d2)

    idx_flat = idx.reshape(EDGES)
    w_flat = w.reshape(EDGES)
    pdeg, offs, vals = _sc_edges(idx_flat, w_flat)

    l_ref = jax.new_ref(jnp.zeros((N * N,), jnp.float32))
    _sc_scatter(l_ref, pdeg, offs, vals)
    return l_ref[...].reshape(N, N)


# speedup vs baseline: 1.4711x; 1.4192x over previous
import functools

import jax
import jax.numpy as jnp
from jax.experimental import pallas as pl
from jax.experimental.pallas import tpu as pltpu

N = 4096
D = 16
KNN = 8
BLK = 512
NBLK = N // BLK


def _knn_kernel(xb_ref, xa_ref, sqb_ref, sqt_ref, d_ref, i_ref, scratch):
    r0 = pl.program_id(0) * BLK
    dot = jax.lax.dot_general(
        xb_ref[...], xa_ref[...], (((1,), (1,)), ((), ())),
        preferred_element_type=jnp.float32,
    )
    d2 = (sqb_ref[...] + sqt_ref[...]) - 2.0 * dot
    d2 = jnp.maximum(d2, 0.0)
    col = jax.lax.broadcasted_iota(jnp.int32, (BLK, N), 1)
    row = jax.lax.broadcasted_iota(jnp.int32, (BLK, N), 0) + r0
    colf = col.astype(jnp.float32)
    inf = jnp.float32(jnp.inf)
    scratch[...] = jnp.where(col == row, inf, d2)
    for k in range(KNN):
        cur = scratch[...]
        mv = jnp.min(cur, axis=1, keepdims=True)
        aminf = jnp.min(jnp.where(cur == mv, colf, jnp.float32(N)),
                        axis=1, keepdims=True)
        d_ref[:, k : k + 1] = mv
        i_ref[:, k : k + 1] = aminf.astype(jnp.int32)
        if k + 1 < KNN:
            scratch[...] = jnp.where(colf == aminf, inf, cur)


def _assemble_kernel(knn_ref, idx_ref, knn_t_ref, idx_t_ref, out_ref):
    r0 = pl.program_id(0) * BLK
    knn_t = knn_t_ref[...]
    sigma = jnp.sum(knn_t) / jnp.float32(N * KNN) + jnp.float32(1e-8)
    w_t = jnp.exp(-knn_t / sigma)
    w_b = jnp.exp(-knn_ref[...] / sigma)
    idx_b = idx_ref[...]

    col = jax.lax.broadcasted_iota(jnp.int32, (BLK, N), 1)
    row = jax.lax.broadcasted_iota(jnp.int32, (BLK, N), 0) + r0
    acc = jnp.zeros((BLK, N), jnp.float32)
    for k in range(KNN):
        own = col == idx_b[:, k : k + 1]
        acc = jnp.where(own, w_b[:, k : k + 1], acc)
        trn = idx_t_ref[k : k + 1, :] == row
        acc = jnp.where(trn, w_t[k : k + 1, :], acc)
    deg = jnp.sum(acc, axis=1, keepdims=True)
    out_ref[...] = jnp.where(col == row, deg, -acc)


@jax.jit
def kernel(x):
    sq = jnp.sum(x * x, axis=1)
    sq_col = sq[:, None]
    sq_row = sq[None, :]

    knn_d2, idx = pl.pallas_call(
        _knn_kernel,
        grid=(NBLK,),
        in_specs=[
            pl.BlockSpec((BLK, D), lambda i: (i, 0)),
            pl.BlockSpec((N, D), lambda i: (0, 0)),
            pl.BlockSpec((BLK, 1), lambda i: (i, 0)),
            pl.BlockSpec((1, N), lambda i: (0, 0)),
        ],
        out_specs=[
            pl.BlockSpec((BLK, KNN), lambda i: (i, 0)),
            pl.BlockSpec((BLK, KNN), lambda i: (i, 0)),
        ],
        out_shape=[
            jax.ShapeDtypeStruct((N, KNN), jnp.float32),
            jax.ShapeDtypeStruct((N, KNN), jnp.int32),
        ],
        scratch_shapes=[pltpu.VMEM((BLK, N), jnp.float32)],
    )(x, x, sq_col, sq_row)

    knn_t = knn_d2.T
    idx_t = idx.T

    L = pl.pallas_call(
        _assemble_kernel,
        grid=(NBLK,),
        in_specs=[
            pl.BlockSpec((BLK, KNN), lambda i: (i, 0)),
            pl.BlockSpec((BLK, KNN), lambda i: (i, 0)),
            pl.BlockSpec((KNN, N), lambda i: (0, 0)),
            pl.BlockSpec((KNN, N), lambda i: (0, 0)),
        ],
        out_specs=pl.BlockSpec((BLK, N), lambda i: (i, 0)),
        out_shape=jax.ShapeDtypeStruct((N, N), jnp.float32),
    )(knn_d2, idx, knn_t, idx_t)
    return L
